# baseline (speedup 1.0000x reference)
import functools

import jax
import jax.numpy as jnp
from jax import lax
from jax.experimental import pallas as pl
from jax.experimental.pallas import tpu as pltpu
from jax.experimental.pallas import tpu_sc as plsc

VOCAB = 1000000
DIM = 64
BATCH = 16384
MAX_NORM = 2.0

_CHUNK = 128
_ROW_BLOCK = 2048


@functools.cache
def _gather_fn():
    info = plsc.get_sparse_core_info()
    nw = info.num_cores * info.num_subcores
    bpw = BATCH // nw
    nchunk = bpw // _CHUNK
    mesh = plsc.VectorSubcoreMesh(core_axis_name="c", subcore_axis_name="s")

    @functools.partial(
        pl.kernel,
        mesh=mesh,
        compiler_params=pltpu.CompilerParams(use_tc_tiling_on_sc=False),
        out_type=[
            jax.ShapeDtypeStruct((BATCH, DIM), jnp.float32),
            jax.ShapeDtypeStruct((BATCH, DIM), jnp.float32),
        ],
        scratch_types=[
            pltpu.VMEM((nchunk, _CHUNK), jnp.int32),
            pltpu.VMEM((nchunk, _CHUNK), jnp.int32),
            pltpu.VMEM((bpw, DIM), jnp.float32),
            pltpu.VMEM((bpw, DIM), jnp.float32),
            pltpu.SemaphoreType.DMA,
            pltpu.SemaphoreType.DMA,
        ],
    )
    def gather(emb_hbm, head_hbm, tail_hbm, hout_hbm, tout_hbm,
               hidx_v, tidx_v, hrows_v, trows_v, hsem, tsem):
        wid = lax.axis_index("s") * info.num_cores + lax.axis_index("c")
        row0 = wid * nchunk
        pltpu.sync_copy(head_hbm.at[pl.ds(row0, nchunk)], hidx_v)
        pltpu.sync_copy(tail_hbm.at[pl.ds(row0, nchunk)], tidx_v)
        hcopies = [
            pltpu.async_copy(emb_hbm.at[hidx_v.at[j]],
                             hrows_v.at[pl.ds(j * _CHUNK, _CHUNK)], hsem)
            for j in range(nchunk)
        ]
        tcopies = [
            pltpu.async_copy(emb_hbm.at[tidx_v.at[j]],
                             trows_v.at[pl.ds(j * _CHUNK, _CHUNK)], tsem)
            for j in range(nchunk)
        ]
        for c in hcopies:
            c.wait()
        base = wid * bpw
        pltpu.sync_copy(hrows_v, hout_hbm.at[pl.ds(base, bpw)])
        for c in tcopies:
            c.wait()
        pltpu.sync_copy(trows_v, tout_hbm.at[pl.ds(base, bpw)])

    return gather


def _mlp_body(h_ref, t_ref, w1h_ref, w1t_ref, b1_ref, w2_ref, b2_ref, o_ref):
    def renorm(v):
        n = jnp.sqrt(jnp.sum(v * v, axis=1, keepdims=True))
        return v * jnp.minimum(1.0, MAX_NORM / jnp.maximum(n, 1e-7))

    h = renorm(h_ref[...])
    t = renorm(t_ref[...])
    acc = jnp.dot(h, w1h_ref[...], preferred_element_type=jnp.float32,
                  precision=lax.Precision.HIGHEST)
    acc += jnp.dot(t, w1t_ref[...], preferred_element_type=jnp.float32,
                   precision=lax.Precision.HIGHEST)
    hid = jnp.tanh(acc + b1_ref[...])
    o_ref[...] = jnp.dot(hid, w2_ref[...], preferred_element_type=jnp.float32,
                         precision=lax.Precision.HIGHEST) + b2_ref[...]


def _mlp(hrows, trows, w1h, w1t, b1, w2, b2):
    grid = (BATCH // _ROW_BLOCK,)
    full = lambda shape: pl.BlockSpec(shape, lambda i: (0, 0))
    return pl.pallas_call(
        _mlp_body,
        grid=grid,
        in_specs=[
            pl.BlockSpec((_ROW_BLOCK, DIM), lambda i: (i, 0)),
            pl.BlockSpec((_ROW_BLOCK, DIM), lambda i: (i, 0)),
            full((DIM, DIM)),
            full((DIM, DIM)),
            full((1, DIM)),
            full((DIM, 2)),
            full((1, 2)),
        ],
        out_specs=pl.BlockSpec((_ROW_BLOCK, 2), lambda i: (i, 0)),
        out_shape=jax.ShapeDtypeStruct((BATCH, 2), jnp.float32),
    )(hrows, trows, w1h, w1t, b1, w2, b2)


def kernel(head, tail, emb, W1, b1, W2, b2):
    head = head.astype(jnp.int32).reshape(BATCH // _CHUNK, _CHUNK)
    tail = tail.astype(jnp.int32).reshape(BATCH // _CHUNK, _CHUNK)
    hrows, trows = _gather_fn()(emb, head, tail)
    return _mlp(hrows, trows, W1[:DIM], W1[DIM:], b1.reshape(1, DIM),
                W2, b2.reshape(1, 2))
